# baseline (speedup 1.0000x reference)
import functools

import jax
import jax.numpy as jnp
from jax import lax
from jax.experimental import pallas as pl
from jax.experimental.pallas import tpu as pltpu
from jax.experimental.pallas import tpu_sc as plsc

K = 10
KP = 16
N = 4096
BS = 4
LW = 128
KNN_R = 256
CONV_R = 2048
DENSE_R = 2048


def _knn_body(xt_ref, xc_ref, oidx_ref):
    xr = xt_ref[0]
    xc = xc_ref[0]
    s = 2.0 * jnp.dot(xr, xc, preferred_element_type=jnp.float32)
    s = s - jnp.sum(xc * xc, axis=0, keepdims=True)
    r, n = s.shape
    col = lax.broadcasted_iota(jnp.int32, (r, n), 1)
    lane = lax.broadcasted_iota(jnp.int32, (r, KP), 1)
    acc = jnp.zeros((r, KP), jnp.int32)
    first = None
    for j in range(K):
        aj = jnp.argmax(s, axis=1).astype(jnp.int32)[:, None]
        if j == 0:
            first = aj
        acc = jnp.where(lane == j, aj, acc)
        s = jnp.where(col == aj, -jnp.inf, s)
    acc = jnp.where(lane >= K, first, acc)
    oidx_ref[0] = acc


def _knn(xt, xc):
    cp = xt.shape[2]
    return pl.pallas_call(
        _knn_body,
        grid=(BS, N // KNN_R),
        in_specs=[
            pl.BlockSpec((1, KNN_R, cp), lambda b, t: (b, t, 0)),
            pl.BlockSpec((1, cp, N), lambda b, t: (b, 0, 0)),
        ],
        out_specs=pl.BlockSpec((1, KNN_R, KP), lambda b, t: (b, t, 0)),
        out_shape=jax.ShapeDtypeStruct((BS, N, KP), jnp.int32),
    )(xt, xc)


def _sc_gather(table, idx):
    b_total = idx.shape[0]
    cp = table.shape[1]
    nw = 32
    bpw = b_total // nw
    ch = 512
    nch = bpw // ch
    mesh = plsc.VectorSubcoreMesh(core_axis_name="c", subcore_axis_name="s")

    @functools.partial(
        pl.kernel, mesh=mesh,
        out_type=jax.ShapeDtypeStruct((b_total, cp), jnp.float32),
        compiler_params=pltpu.CompilerParams(use_tc_tiling_on_sc=False),
        scratch_types=[
            pltpu.VMEM((ch,), jnp.int32),
            pltpu.VMEM((ch, cp), jnp.float32),
            pltpu.SemaphoreType.DMA,
        ],
    )
    def k(table_hbm, idx_hbm, out_hbm, idx_v, rows_v, sem):
        wid = lax.axis_index("s") * 2 + lax.axis_index("c")
        base = wid * bpw
        for ci in range(nch):
            off = base + ci * ch
            pltpu.sync_copy(idx_hbm.at[pl.ds(off, ch)], idx_v)
            pltpu.async_copy(table_hbm.at[idx_v], rows_v, sem).wait()
            pltpu.sync_copy(rows_v, out_hbm.at[pl.ds(off, ch)])

    return k(table, idx)


def _zcompute(nb_ref, t_ref, wc_ref, wr_ref):
    t = t_ref[...]
    t = jnp.broadcast_to(t[:, None, :], (t.shape[0], KP, t.shape[1]))
    cen = t.reshape(nb_ref.shape[0], t.shape[2])
    nb = nb_ref[...]
    return (jnp.dot(nb - cen, wc_ref[...], preferred_element_type=jnp.float32)
            + jnp.dot(cen, wr_ref[...], preferred_element_type=jnp.float32))


def _masked_stats(z):
    row = lax.broadcasted_iota(jnp.int32, (z.shape[0], 1), 0)
    m = jnp.where(row % KP < K, 1.0, 0.0)
    zm = z * m
    s = jnp.sum(zm, axis=0, keepdims=True)
    ss = jnp.sum(zm * z, axis=0, keepdims=True)
    return jnp.concatenate([s, ss], axis=0)


def _lrelu(y):
    return jnp.where(y >= 0, y, 0.2 * y)


def _affine(s_ref, nelt, g_ref, e_ref):
    mean = s_ref[0:1, :] * (1.0 / nelt)
    var = s_ref[1:2, :] * (1.0 / nelt) - mean * mean
    sc = g_ref[...] / jnp.sqrt(var + 1e-5)
    return sc, e_ref[...] - mean * sc


def _conv_fused_body(nb_ref, t_ref, wc_ref, wr_ref, wb_ref, ga_ref, ea_ref,
                     gb_ref, eb_ref, o_ref, sa_ref, sb_ref):
    p = pl.program_id(0)
    t = pl.program_id(1)
    nelt = float(BS * N * K)
    z = _zcompute(nb_ref, t_ref, wc_ref, wr_ref)

    @pl.when(jnp.logical_and(p == 0, t == 0))
    def _():
        sa_ref[...] = jnp.zeros_like(sa_ref)

    @pl.when(jnp.logical_and(p == 1, t == 0))
    def _():
        sb_ref[...] = jnp.zeros_like(sb_ref)

    @pl.when(p == 0)
    def _():
        sa_ref[...] += _masked_stats(z)

    @pl.when(p > 0)
    def _():
        sc_a, sh_a = _affine(sa_ref, nelt, ga_ref, ea_ref)
        act = _lrelu(z * sc_a + sh_a)
        z2 = jnp.dot(act, wb_ref[...], preferred_element_type=jnp.float32)

        @pl.when(p == 1)
        def _():
            sb_ref[...] += _masked_stats(z2)

        @pl.when(p == 2)
        def _():
            sc_b, sh_b = _affine(sb_ref, nelt, gb_ref, eb_ref)
            act2 = _lrelu(z2 * sc_b + sh_b)
            np_, ob = act2.shape[0] // KP, act2.shape[1]
            o_ref[...] = jnp.max(act2.reshape(np_, KP, ob), axis=1)


def _conv_block(nb, tmat, wc, wr, wb, ga, ea, gb, eb):
    bt, cp = nb.shape
    npnt = tmat.shape[0]
    oa, ob = wc.shape[1], wb.shape[1]
    nb_spec = pl.BlockSpec((CONV_R, cp), lambda p, t: (t, 0))
    t_spec = pl.BlockSpec((CONV_R // KP, cp), lambda p, t: (t, 0))

    def cs(arr):
        return pl.BlockSpec(arr.shape, lambda p, t: (0,) * arr.ndim)

    ntiles = bt // CONV_R
    out = pl.pallas_call(
        _conv_fused_body, grid=(3, ntiles),
        in_specs=[nb_spec, t_spec, cs(wc), cs(wr), cs(wb),
                  cs(ga.reshape(1, oa)), cs(ea.reshape(1, oa)),
                  cs(gb.reshape(1, ob)), cs(eb.reshape(1, ob))],
        out_specs=pl.BlockSpec((CONV_R // KP, ob),
                               lambda p, t: (jnp.where(p == 2, t, ntiles), 0)),
        out_shape=jax.ShapeDtypeStruct((npnt + CONV_R // KP, ob), jnp.float32),
        scratch_shapes=[pltpu.VMEM((2, oa), jnp.float32),
                        pltpu.VMEM((2, ob), jnp.float32)],
    )(nb, tmat, wc, wr, wb, ga.reshape(1, oa), ea.reshape(1, oa),
      gb.reshape(1, ob), eb.reshape(1, ob))
    return out[:npnt]


def _stats(z):
    s = jnp.sum(z, axis=0, keepdims=True)
    ss = jnp.sum(z * z, axis=0, keepdims=True)
    return jnp.concatenate([s, ss], axis=0)


def _dense_fused_body(h_ref, w_ref, w2_ref, ga_ref, ea_ref, gb_ref, eb_ref,
                      o_ref, sa_ref, sb_ref):
    p = pl.program_id(0)
    t = pl.program_id(1)
    nelt = float(BS * N)
    z = jnp.dot(h_ref[...], w_ref[...], preferred_element_type=jnp.float32)

    @pl.when(jnp.logical_and(p == 0, t == 0))
    def _():
        sa_ref[...] = jnp.zeros_like(sa_ref)

    @pl.when(jnp.logical_and(p == 1, t == 0))
    def _():
        sb_ref[...] = jnp.zeros_like(sb_ref)

    @pl.when(p == 0)
    def _():
        sa_ref[...] += _stats(z)

    @pl.when(p > 0)
    def _():
        sc_a, sh_a = _affine(sa_ref, nelt, ga_ref, ea_ref)
        act = _lrelu(z * sc_a + sh_a)
        z2 = jnp.dot(act, w2_ref[...], preferred_element_type=jnp.float32)

        @pl.when(p == 1)
        def _():
            sb_ref[...] += _stats(z2)

        @pl.when(p == 2)
        def _():
            sc_b, sh_b = _affine(sb_ref, nelt, gb_ref, eb_ref)
            o_ref[...] = _lrelu(z2 * sc_b + sh_b)


def _dense_head(h, w1, g1, e1, w2, g2, e2):
    nr, ci = h.shape
    o1, o2 = w1.shape[1], w2.shape[1]
    h_spec = pl.BlockSpec((DENSE_R, ci), lambda p, t: (t, 0))

    def cs(arr):
        return pl.BlockSpec(arr.shape, lambda p, t: (0,) * arr.ndim)

    ntiles = nr // DENSE_R
    out = pl.pallas_call(
        _dense_fused_body, grid=(3, ntiles),
        in_specs=[h_spec, cs(w1), cs(w2),
                  cs(g1.reshape(1, o1)), cs(e1.reshape(1, o1)),
                  cs(g2.reshape(1, o2)), cs(e2.reshape(1, o2))],
        out_specs=pl.BlockSpec((DENSE_R, o2),
                               lambda p, t: (jnp.where(p == 2, t, ntiles), 0)),
        out_shape=jax.ShapeDtypeStruct((nr + DENSE_R, o2), jnp.float32),
        scratch_shapes=[pltpu.VMEM((2, o1), jnp.float32),
                        pltpu.VMEM((2, o2), jnp.float32)],
    )(h, w1, w2, g1.reshape(1, o1), e1.reshape(1, o1),
      g2.reshape(1, o2), e2.reshape(1, o2))
    return out[:nr]


def _split_w(w, c, cp):
    pad = ((0, cp - c), (0, 0))
    return jnp.pad(w[:, :c].T, pad), jnp.pad(w[:, c:].T, pad)


def _graph_block(x_rows, c, cp, wa, ga, ea, wb, gb, eb):
    xp = jnp.pad(x_rows, ((0, 0), (0, cp - c)))
    xt = xp.reshape(BS, N, cp)
    xc = jnp.transpose(xt, (0, 2, 1))
    idx = _knn(xt, xc)
    gidx = (idx + (jnp.arange(BS, dtype=jnp.int32) * N)[:, None, None])
    nb = _sc_gather(xp, gidx.reshape(-1))
    wc, wr = _split_w(wa, c, cp)
    return _conv_block(nb, xp, wc, wr, wb.T, ga, ea, gb, eb)


def kernel(x, W1a, b1a, g1a, e1a, W1b, b1b, g1b, e1b,
           W2a, b2a, g2a, e2a, W2b, b2b, g2b, e2b,
           W3a, b3a, g3a, e3a, W3b, b3b, g3b, e3b):
    del b1a, b1b, b2a, b2b, b3a, b3b
    x_rows = jnp.transpose(x, (0, 2, 1)).reshape(BS * N, 3)
    x1 = _graph_block(x_rows, 3, 16, W1a, g1a, e1a, W1b, g1b, e1b)
    x2 = _graph_block(x1, 19, 32, W2a, g2a, e2a, W2b, g2b, e2b)
    h = jnp.concatenate([x1, x2], axis=1)
    out = _dense_head(h, W3a.T, g3a, e3a, W3b.T, g3b, e3b)
    return jnp.transpose(out.reshape(BS, N, 128), (0, 2, 1))


# speedup vs baseline: 1.0490x; 1.0490x over previous
import functools

import jax
import jax.numpy as jnp
from jax import lax
from jax.experimental import pallas as pl
from jax.experimental.pallas import tpu as pltpu
from jax.experimental.pallas import tpu_sc as plsc

K = 10
KP = 16
N = 4096
BS = 4
LW = 128
KNN_R = 256
CONV_R = 2048
DENSE_R = 2048


def _knn_body(xt_ref, xc_ref, oidx_ref):
    xr = xt_ref[0]
    xc = xc_ref[0]
    s = 2.0 * jnp.dot(xr, xc, preferred_element_type=jnp.float32)
    s = s - jnp.sum(xc * xc, axis=0, keepdims=True)
    r, n = s.shape
    col = lax.broadcasted_iota(jnp.int32, (r, n), 1)
    lane = lax.broadcasted_iota(jnp.int32, (r, KP), 1)
    acc = jnp.zeros((r, KP), jnp.int32)
    first = None
    for j in range(K):
        aj = jnp.argmax(s, axis=1).astype(jnp.int32)[:, None]
        if j == 0:
            first = aj
        acc = jnp.where(lane == j, aj, acc)
        s = jnp.where(col == aj, -jnp.inf, s)
    acc = jnp.where(lane >= K, first, acc)
    oidx_ref[0] = acc


def _knn(xt, xc):
    cp = xt.shape[2]
    return pl.pallas_call(
        _knn_body,
        grid=(BS, N // KNN_R),
        in_specs=[
            pl.BlockSpec((1, KNN_R, cp), lambda b, t: (b, t, 0)),
            pl.BlockSpec((1, cp, N), lambda b, t: (b, 0, 0)),
        ],
        out_specs=pl.BlockSpec((1, KNN_R, KP), lambda b, t: (b, t, 0)),
        out_shape=jax.ShapeDtypeStruct((BS, N, KP), jnp.int32),
    )(xt, xc)


def _sc_gather(table, idx):
    b_total = idx.shape[0]
    cp = table.shape[1]
    nw = 32
    bpw = b_total // nw
    ch = 512
    nch = bpw // ch
    mesh = plsc.VectorSubcoreMesh(core_axis_name="c", subcore_axis_name="s")

    @functools.partial(
        pl.kernel, mesh=mesh,
        out_type=jax.ShapeDtypeStruct((b_total, cp), jnp.float32),
        compiler_params=pltpu.CompilerParams(use_tc_tiling_on_sc=False),
        scratch_types=[
            pltpu.VMEM((ch,), jnp.int32),
            pltpu.VMEM((ch, cp), jnp.float32),
            pltpu.SemaphoreType.DMA,
        ],
    )
    def k(table_hbm, idx_hbm, out_hbm, idx_v, rows_v, sem):
        wid = lax.axis_index("s") * 2 + lax.axis_index("c")
        base = wid * bpw
        for ci in range(nch):
            off = base + ci * ch
            pltpu.sync_copy(idx_hbm.at[pl.ds(off, ch)], idx_v)
            pltpu.async_copy(table_hbm.at[idx_v], rows_v, sem).wait()
            pltpu.sync_copy(rows_v, out_hbm.at[pl.ds(off, ch)])

    return k(table, idx)


def _zcompute(nb_ref, t_ref, wc_ref, wr_ref):
    t = t_ref[...]
    t = jnp.broadcast_to(t[:, None, :], (t.shape[0], KP, t.shape[1]))
    cen = t.reshape(nb_ref.shape[0], t.shape[2])
    nb = nb_ref[...]
    return (jnp.dot(nb - cen, wc_ref[...], preferred_element_type=jnp.float32)
            + jnp.dot(cen, wr_ref[...], preferred_element_type=jnp.float32))


def _masked_stats(z):
    row = lax.broadcasted_iota(jnp.int32, (z.shape[0], 1), 0)
    m = jnp.where(row % KP < K, 1.0, 0.0)
    zm = z * m
    s = jnp.sum(zm, axis=0, keepdims=True)
    ss = jnp.sum(zm * z, axis=0, keepdims=True)
    return jnp.concatenate([s, ss], axis=0)


def _lrelu(y):
    return jnp.where(y >= 0, y, 0.2 * y)


def _acc_out(o_ref, val):
    @pl.when(pl.program_id(0) == 0)
    def _():
        o_ref[...] = jnp.zeros_like(o_ref)
    o_ref[...] += val


def _const_spec(arr):
    return pl.BlockSpec(arr.shape, lambda t: (0,) * arr.ndim)


def _bn_affine(sums, nelt, g, e):
    o = g.shape[0]
    mean = sums[0] / nelt
    var = sums[1] / nelt - mean * mean
    sc = g / jnp.sqrt(var + 1e-5)
    sh = e - mean * sc
    return sc.reshape(1, o), sh.reshape(1, o)


def _conv_stats_a_body(nb_ref, t_ref, wc_ref, wr_ref, o_ref):
    _acc_out(o_ref, _masked_stats(_zcompute(nb_ref, t_ref, wc_ref, wr_ref)))


def _conv_stats_b_body(nb_ref, t_ref, wc_ref, wr_ref, sc_ref, sh_ref, wb_ref,
                       o_ref):
    z = _zcompute(nb_ref, t_ref, wc_ref, wr_ref)
    act = _lrelu(z * sc_ref[...] + sh_ref[...])
    z2 = jnp.dot(act, wb_ref[...], preferred_element_type=jnp.float32)
    _acc_out(o_ref, _masked_stats(z2))


def _conv_final_body(nb_ref, t_ref, wc_ref, wr_ref, sc_ref, sh_ref, wb_ref,
                     sc2_ref, sh2_ref, o_ref):
    z = _zcompute(nb_ref, t_ref, wc_ref, wr_ref)
    act = _lrelu(z * sc_ref[...] + sh_ref[...])
    z2 = jnp.dot(act, wb_ref[...], preferred_element_type=jnp.float32)
    act2 = _lrelu(z2 * sc2_ref[...] + sh2_ref[...])
    np_, ob = act2.shape[0] // KP, act2.shape[1]
    o_ref[...] = jnp.max(act2.reshape(np_, KP, ob), axis=1)


def _conv_block(nb, tmat, wc, wr, wb, ga, ea, gb, eb):
    bt, cp = nb.shape
    npnt = tmat.shape[0]
    oa, ob = wc.shape[1], wb.shape[1]
    nb_spec = pl.BlockSpec((CONV_R, cp), lambda t: (t, 0))
    t_spec = pl.BlockSpec((CONV_R // KP, cp), lambda t: (t, 0))

    grid = (bt // CONV_R,)
    nelt = BS * N * K

    sums_a = pl.pallas_call(
        _conv_stats_a_body, grid=grid,
        in_specs=[nb_spec, t_spec, _const_spec(wc), _const_spec(wr)],
        out_specs=pl.BlockSpec((2, oa), lambda t: (0, 0)),
        out_shape=jax.ShapeDtypeStruct((2, oa), jnp.float32),
    )(nb, tmat, wc, wr)
    sc_a, sh_a = _bn_affine(sums_a, nelt, ga, ea)

    sums_b = pl.pallas_call(
        _conv_stats_b_body, grid=grid,
        in_specs=[nb_spec, t_spec, _const_spec(wc), _const_spec(wr),
                  _const_spec(sc_a), _const_spec(sh_a), _const_spec(wb)],
        out_specs=pl.BlockSpec((2, ob), lambda t: (0, 0)),
        out_shape=jax.ShapeDtypeStruct((2, ob), jnp.float32),
    )(nb, tmat, wc, wr, sc_a, sh_a, wb)
    sc_b, sh_b = _bn_affine(sums_b, nelt, gb, eb)

    return pl.pallas_call(
        _conv_final_body, grid=grid,
        in_specs=[nb_spec, t_spec, _const_spec(wc), _const_spec(wr),
                  _const_spec(sc_a), _const_spec(sh_a), _const_spec(wb),
                  _const_spec(sc_b), _const_spec(sh_b)],
        out_specs=pl.BlockSpec((CONV_R // KP, ob), lambda t: (t, 0)),
        out_shape=jax.ShapeDtypeStruct((npnt, ob), jnp.float32),
    )(nb, tmat, wc, wr, sc_a, sh_a, wb, sc_b, sh_b)


def _stats(z):
    s = jnp.sum(z, axis=0, keepdims=True)
    ss = jnp.sum(z * z, axis=0, keepdims=True)
    return jnp.concatenate([s, ss], axis=0)


def _dense_stats_a_body(h_ref, w_ref, o_ref):
    z = jnp.dot(h_ref[...], w_ref[...], preferred_element_type=jnp.float32)
    _acc_out(o_ref, _stats(z))


def _dense_stats_b_body(h_ref, w_ref, sc_ref, sh_ref, w2_ref, o_ref):
    z = jnp.dot(h_ref[...], w_ref[...], preferred_element_type=jnp.float32)
    act = _lrelu(z * sc_ref[...] + sh_ref[...])
    z2 = jnp.dot(act, w2_ref[...], preferred_element_type=jnp.float32)
    _acc_out(o_ref, _stats(z2))


def _dense_final_body(h_ref, w_ref, sc_ref, sh_ref, w2_ref, sc2_ref, sh2_ref,
                      o_ref):
    z = jnp.dot(h_ref[...], w_ref[...], preferred_element_type=jnp.float32)
    act = _lrelu(z * sc_ref[...] + sh_ref[...])
    z2 = jnp.dot(act, w2_ref[...], preferred_element_type=jnp.float32)
    o_ref[...] = _lrelu(z2 * sc2_ref[...] + sh2_ref[...])


def _dense_head(h, w1, g1, e1, w2, g2, e2):
    nr, ci = h.shape
    o1, o2 = w1.shape[1], w2.shape[1]
    grid = (nr // DENSE_R,)
    h_spec = pl.BlockSpec((DENSE_R, ci), lambda t: (t, 0))

    sums = pl.pallas_call(
        _dense_stats_a_body, grid=grid,
        in_specs=[h_spec, _const_spec(w1)],
        out_specs=pl.BlockSpec((2, o1), lambda t: (0, 0)),
        out_shape=jax.ShapeDtypeStruct((2, o1), jnp.float32),
    )(h, w1)
    sc1, sh1 = _bn_affine(sums, nr, g1, e1)

    sums2 = pl.pallas_call(
        _dense_stats_b_body, grid=grid,
        in_specs=[h_spec, _const_spec(w1), _const_spec(sc1), _const_spec(sh1),
                  _const_spec(w2)],
        out_specs=pl.BlockSpec((2, o2), lambda t: (0, 0)),
        out_shape=jax.ShapeDtypeStruct((2, o2), jnp.float32),
    )(h, w1, sc1, sh1, w2)
    sc2, sh2 = _bn_affine(sums2, nr, g2, e2)

    return pl.pallas_call(
        _dense_final_body, grid=grid,
        in_specs=[h_spec, _const_spec(w1), _const_spec(sc1), _const_spec(sh1),
                  _const_spec(w2), _const_spec(sc2), _const_spec(sh2)],
        out_specs=pl.BlockSpec((DENSE_R, o2), lambda t: (t, 0)),
        out_shape=jax.ShapeDtypeStruct((nr, o2), jnp.float32),
    )(h, w1, sc1, sh1, w2, sc2, sh2)


def _split_w(w, c, cp):
    pad = ((0, cp - c), (0, 0))
    return jnp.pad(w[:, :c].T, pad), jnp.pad(w[:, c:].T, pad)


def _graph_block(x_rows, c, cp, wa, ga, ea, wb, gb, eb):
    xp = jnp.pad(x_rows, ((0, 0), (0, cp - c)))
    xt = xp.reshape(BS, N, cp)
    xc = jnp.transpose(xt, (0, 2, 1))
    idx = _knn(xt, xc)
    gidx = (idx + (jnp.arange(BS, dtype=jnp.int32) * N)[:, None, None])
    nb = _sc_gather(xp, gidx.reshape(-1))
    wc, wr = _split_w(wa, c, cp)
    return _conv_block(nb, xp, wc, wr, wb.T, ga, ea, gb, eb)


def kernel(x, W1a, b1a, g1a, e1a, W1b, b1b, g1b, e1b,
           W2a, b2a, g2a, e2a, W2b, b2b, g2b, e2b,
           W3a, b3a, g3a, e3a, W3b, b3b, g3b, e3b):
    del b1a, b1b, b2a, b2b, b3a, b3b
    x_rows = jnp.transpose(x, (0, 2, 1)).reshape(BS * N, 3)
    x1 = _graph_block(x_rows, 3, 16, W1a, g1a, e1a, W1b, g1b, e1b)
    x2 = _graph_block(x1, 19, 32, W2a, g2a, e2a, W2b, g2b, e2b)
    h = jnp.concatenate([x1, x2], axis=1)
    out = _dense_head(h, W3a.T, g3a, e3a, W3b.T, g3b, e3b)
    return jnp.transpose(out.reshape(BS, N, 128), (0, 2, 1))


# speedup vs baseline: 1.0653x; 1.0155x over previous
import functools

import jax
import jax.numpy as jnp
from jax import lax
from jax.experimental import pallas as pl
from jax.experimental.pallas import tpu as pltpu
from jax.experimental.pallas import tpu_sc as plsc

K = 10
KP = 16
N = 4096
BS = 4
LW = 128
KNN_R = 512
CONV_R = 2048
DENSE_R = 2048


def _knn_body(xt_ref, xc_ref, oidx_ref):
    xr = xt_ref[0]
    xc = xc_ref[0]
    s = 2.0 * jnp.dot(xr, xc, preferred_element_type=jnp.float32)
    s = s - jnp.sum(xc * xc, axis=0, keepdims=True)
    r, n = s.shape
    col = lax.broadcasted_iota(jnp.int32, (r, n), 1)
    lane = lax.broadcasted_iota(jnp.int32, (r, KP), 1)
    acc = jnp.zeros((r, KP), jnp.int32)
    first = None
    for j in range(K):
        aj = jnp.argmax(s, axis=1).astype(jnp.int32)[:, None]
        if j == 0:
            first = aj
        acc = jnp.where(lane == j, aj, acc)
        s = jnp.where(col == aj, -jnp.inf, s)
    acc = jnp.where(lane >= K, first, acc)
    oidx_ref[0] = acc


def _knn(xt, xc):
    cp = xt.shape[2]
    return pl.pallas_call(
        _knn_body,
        grid=(BS, N // KNN_R),
        in_specs=[
            pl.BlockSpec((1, KNN_R, cp), lambda b, t: (b, t, 0)),
            pl.BlockSpec((1, cp, N), lambda b, t: (b, 0, 0)),
        ],
        out_specs=pl.BlockSpec((1, KNN_R, KP), lambda b, t: (b, t, 0)),
        out_shape=jax.ShapeDtypeStruct((BS, N, KP), jnp.int32),
    )(xt, xc)


def _sc_gather(table, idx):
    b_total = idx.shape[0]
    cp = table.shape[1]
    nw = 32
    bpw = b_total // nw
    ch = 512
    nch = bpw // ch
    mesh = plsc.VectorSubcoreMesh(core_axis_name="c", subcore_axis_name="s")

    @functools.partial(
        pl.kernel, mesh=mesh,
        out_type=jax.ShapeDtypeStruct((b_total, cp), jnp.float32),
        compiler_params=pltpu.CompilerParams(use_tc_tiling_on_sc=False),
        scratch_types=[
            pltpu.VMEM((ch,), jnp.int32),
            pltpu.VMEM((ch, cp), jnp.float32),
            pltpu.SemaphoreType.DMA,
        ],
    )
    def k(table_hbm, idx_hbm, out_hbm, idx_v, rows_v, sem):
        wid = lax.axis_index("s") * 2 + lax.axis_index("c")
        base = wid * bpw
        for ci in range(nch):
            off = base + ci * ch
            pltpu.sync_copy(idx_hbm.at[pl.ds(off, ch)], idx_v)
            pltpu.async_copy(table_hbm.at[idx_v], rows_v, sem).wait()
            pltpu.sync_copy(rows_v, out_hbm.at[pl.ds(off, ch)])

    return k(table, idx)


def _zcompute(nb_ref, t_ref, wc_ref, wr_ref):
    t = t_ref[...]
    t = jnp.broadcast_to(t[:, None, :], (t.shape[0], KP, t.shape[1]))
    cen = t.reshape(nb_ref.shape[0], t.shape[2])
    nb = nb_ref[...]
    return (jnp.dot(nb - cen, wc_ref[...], preferred_element_type=jnp.float32)
            + jnp.dot(cen, wr_ref[...], preferred_element_type=jnp.float32))


def _masked_stats(z):
    row = lax.broadcasted_iota(jnp.int32, (z.shape[0], 1), 0)
    m = jnp.where(row % KP < K, 1.0, 0.0)
    zm = z * m
    s = jnp.sum(zm, axis=0, keepdims=True)
    ss = jnp.sum(zm * z, axis=0, keepdims=True)
    return jnp.concatenate([s, ss], axis=0)


def _lrelu(y):
    return jnp.where(y >= 0, y, 0.2 * y)


def _acc_out(o_ref, val):
    @pl.when(pl.program_id(0) == 0)
    def _():
        o_ref[...] = jnp.zeros_like(o_ref)
    o_ref[...] += val


def _const_spec(arr):
    return pl.BlockSpec(arr.shape, lambda t: (0,) * arr.ndim)


def _bn_affine(sums, nelt, g, e):
    o = g.shape[0]
    mean = sums[0] / nelt
    var = sums[1] / nelt - mean * mean
    sc = g / jnp.sqrt(var + 1e-5)
    sh = e - mean * sc
    return sc.reshape(1, o), sh.reshape(1, o)


def _conv_stats_a_body(nb_ref, t_ref, wc_ref, wr_ref, o_ref):
    _acc_out(o_ref, _masked_stats(_zcompute(nb_ref, t_ref, wc_ref, wr_ref)))


def _conv_stats_b_body(nb_ref, t_ref, wc_ref, wr_ref, sc_ref, sh_ref, wb_ref,
                       o_ref):
    z = _zcompute(nb_ref, t_ref, wc_ref, wr_ref)
    act = _lrelu(z * sc_ref[...] + sh_ref[...])
    z2 = jnp.dot(act, wb_ref[...], preferred_element_type=jnp.float32)
    _acc_out(o_ref, _masked_stats(z2))


def _conv_final_body(nb_ref, t_ref, wc_ref, wr_ref, sc_ref, sh_ref, wb_ref,
                     sc2_ref, sh2_ref, o_ref):
    z = _zcompute(nb_ref, t_ref, wc_ref, wr_ref)
    act = _lrelu(z * sc_ref[...] + sh_ref[...])
    z2 = jnp.dot(act, wb_ref[...], preferred_element_type=jnp.float32)
    act2 = _lrelu(z2 * sc2_ref[...] + sh2_ref[...])
    np_, ob = act2.shape[0] // KP, act2.shape[1]
    o_ref[...] = jnp.max(act2.reshape(np_, KP, ob), axis=1)


def _conv_block(nb, tmat, wc, wr, wb, ga, ea, gb, eb):
    bt, cp = nb.shape
    npnt = tmat.shape[0]
    oa, ob = wc.shape[1], wb.shape[1]
    nb_spec = pl.BlockSpec((CONV_R, cp), lambda t: (t, 0))
    t_spec = pl.BlockSpec((CONV_R // KP, cp), lambda t: (t, 0))

    grid = (bt // CONV_R,)
    nelt = BS * N * K

    sums_a = pl.pallas_call(
        _conv_stats_a_body, grid=grid,
        in_specs=[nb_spec, t_spec, _const_spec(wc), _const_spec(wr)],
        out_specs=pl.BlockSpec((2, oa), lambda t: (0, 0)),
        out_shape=jax.ShapeDtypeStruct((2, oa), jnp.float32),
    )(nb, tmat, wc, wr)
    sc_a, sh_a = _bn_affine(sums_a, nelt, ga, ea)

    sums_b = pl.pallas_call(
        _conv_stats_b_body, grid=grid,
        in_specs=[nb_spec, t_spec, _const_spec(wc), _const_spec(wr),
                  _const_spec(sc_a), _const_spec(sh_a), _const_spec(wb)],
        out_specs=pl.BlockSpec((2, ob), lambda t: (0, 0)),
        out_shape=jax.ShapeDtypeStruct((2, ob), jnp.float32),
    )(nb, tmat, wc, wr, sc_a, sh_a, wb)
    sc_b, sh_b = _bn_affine(sums_b, nelt, gb, eb)

    return pl.pallas_call(
        _conv_final_body, grid=grid,
        in_specs=[nb_spec, t_spec, _const_spec(wc), _const_spec(wr),
                  _const_spec(sc_a), _const_spec(sh_a), _const_spec(wb),
                  _const_spec(sc_b), _const_spec(sh_b)],
        out_specs=pl.BlockSpec((CONV_R // KP, ob), lambda t: (t, 0)),
        out_shape=jax.ShapeDtypeStruct((npnt, ob), jnp.float32),
    )(nb, tmat, wc, wr, sc_a, sh_a, wb, sc_b, sh_b)


def _stats(z):
    s = jnp.sum(z, axis=0, keepdims=True)
    ss = jnp.sum(z * z, axis=0, keepdims=True)
    return jnp.concatenate([s, ss], axis=0)


def _dense_stats_a_body(h_ref, w_ref, o_ref):
    z = jnp.dot(h_ref[...], w_ref[...], preferred_element_type=jnp.float32)
    _acc_out(o_ref, _stats(z))


def _dense_stats_b_body(h_ref, w_ref, sc_ref, sh_ref, w2_ref, o_ref):
    z = jnp.dot(h_ref[...], w_ref[...], preferred_element_type=jnp.float32)
    act = _lrelu(z * sc_ref[...] + sh_ref[...])
    z2 = jnp.dot(act, w2_ref[...], preferred_element_type=jnp.float32)
    _acc_out(o_ref, _stats(z2))


def _dense_final_body(h_ref, w_ref, sc_ref, sh_ref, w2_ref, sc2_ref, sh2_ref,
                      o_ref):
    z = jnp.dot(h_ref[...], w_ref[...], preferred_element_type=jnp.float32)
    act = _lrelu(z * sc_ref[...] + sh_ref[...])
    z2 = jnp.dot(act, w2_ref[...], preferred_element_type=jnp.float32)
    o_ref[...] = _lrelu(z2 * sc2_ref[...] + sh2_ref[...])


def _dense_head(h, w1, g1, e1, w2, g2, e2):
    nr, ci = h.shape
    o1, o2 = w1.shape[1], w2.shape[1]
    grid = (nr // DENSE_R,)
    h_spec = pl.BlockSpec((DENSE_R, ci), lambda t: (t, 0))

    sums = pl.pallas_call(
        _dense_stats_a_body, grid=grid,
        in_specs=[h_spec, _const_spec(w1)],
        out_specs=pl.BlockSpec((2, o1), lambda t: (0, 0)),
        out_shape=jax.ShapeDtypeStruct((2, o1), jnp.float32),
    )(h, w1)
    sc1, sh1 = _bn_affine(sums, nr, g1, e1)

    sums2 = pl.pallas_call(
        _dense_stats_b_body, grid=grid,
        in_specs=[h_spec, _const_spec(w1), _const_spec(sc1), _const_spec(sh1),
                  _const_spec(w2)],
        out_specs=pl.BlockSpec((2, o2), lambda t: (0, 0)),
        out_shape=jax.ShapeDtypeStruct((2, o2), jnp.float32),
    )(h, w1, sc1, sh1, w2)
    sc2, sh2 = _bn_affine(sums2, nr, g2, e2)

    return pl.pallas_call(
        _dense_final_body, grid=grid,
        in_specs=[h_spec, _const_spec(w1), _const_spec(sc1), _const_spec(sh1),
                  _const_spec(w2), _const_spec(sc2), _const_spec(sh2)],
        out_specs=pl.BlockSpec((DENSE_R, o2), lambda t: (t, 0)),
        out_shape=jax.ShapeDtypeStruct((nr, o2), jnp.float32),
    )(h, w1, sc1, sh1, w2, sc2, sh2)


def _split_w(w, c, cp):
    pad = ((0, cp - c), (0, 0))
    return jnp.pad(w[:, :c].T, pad), jnp.pad(w[:, c:].T, pad)


def _graph_block(x_rows, c, cp, wa, ga, ea, wb, gb, eb):
    xp = jnp.pad(x_rows, ((0, 0), (0, cp - c)))
    xt = xp.reshape(BS, N, cp)
    xc = jnp.transpose(xt, (0, 2, 1))
    idx = _knn(xt, xc)
    gidx = (idx + (jnp.arange(BS, dtype=jnp.int32) * N)[:, None, None])
    nb = _sc_gather(xp, gidx.reshape(-1))
    wc, wr = _split_w(wa, c, cp)
    return _conv_block(nb, xp, wc, wr, wb.T, ga, ea, gb, eb)


def kernel(x, W1a, b1a, g1a, e1a, W1b, b1b, g1b, e1b,
           W2a, b2a, g2a, e2a, W2b, b2b, g2b, e2b,
           W3a, b3a, g3a, e3a, W3b, b3b, g3b, e3b):
    del b1a, b1b, b2a, b2b, b3a, b3b
    x_rows = jnp.transpose(x, (0, 2, 1)).reshape(BS * N, 3)
    x1 = _graph_block(x_rows, 3, 16, W1a, g1a, e1a, W1b, g1b, e1b)
    x2 = _graph_block(x1, 19, 32, W2a, g2a, e2a, W2b, g2b, e2b)
    h = jnp.concatenate([x1, x2], axis=1)
    out = _dense_head(h, W3a.T, g3a, e3a, W3b.T, g3b, e3b)
    return jnp.transpose(out.reshape(BS, N, 128), (0, 2, 1))
